# baseline (speedup 1.0000x reference)
import functools

import jax
import jax.numpy as jnp
from jax import lax
from jax.experimental import pallas as pl
from jax.experimental.pallas import tpu as pltpu
from jax.experimental.pallas import tpu_sc as plsc

B = 128
L = 77
D_IN = 512
D_EMB = 1024
HID = 512
N_SEL = 10
N_ATT = 75
PAD_ATT = 80
NC, NS = 2, 16
NW = NC * NS
SPW = B // NW
SLAB = 128
NSTEP = B // SLAB

_NEG = float("-inf")
_BIG = 2**30


def _lane_rot(x, r):
    perm = (lax.iota(jnp.int32, 16) + r) & 15
    dn = lax.GatherDimensionNumbers(
        offset_dims=(), collapsed_slice_dims=(0,), start_index_map=(0,))
    return lax.gather(x, perm[:, None], dn, slice_sizes=(1,),
                      mode=lax.GatherScatterMode.PROMISE_IN_BOUNDS)


def _all_lanes_reduce(x, op):
    for r in (8, 4, 2, 1):
        x = op(x, _lane_rot(x, r))
    return x


def _sc_topk(att_pad):
    mesh = plsc.VectorSubcoreMesh(core_axis_name="c", subcore_axis_name="s")

    @functools.partial(
        pl.kernel,
        mesh=mesh,
        out_type=jax.ShapeDtypeStruct((B, 16), jnp.int32),
        scratch_types=[
            pltpu.VMEM((SPW, PAD_ATT), jnp.float32),
            pltpu.VMEM((SPW, 16), jnp.int32),
        ],
    )
    def k(att_hbm, out_hbm, att_v, idx_v):
        wid = lax.axis_index("s") * NC + lax.axis_index("c")
        s0 = wid * SPW
        pltpu.sync_copy(att_hbm.at[pl.ds(s0, SPW)], att_v)
        iota = lax.iota(jnp.int32, 16)
        neg = jnp.full((16,), _NEG, jnp.float32)
        for j in range(SPW):
            chunks = [att_v[j, pl.ds(ci * 16, 16)] for ci in range(PAD_ATT // 16)]
            idxvs = [iota + ci * 16 for ci in range(PAD_ATT // 16)]
            chosen = jnp.zeros((16,), jnp.int32)
            for t in range(N_SEL):
                m = chunks[0]
                for c in chunks[1:]:
                    m = jnp.maximum(m, c)
                s = _all_lanes_reduce(m, jnp.maximum)
                cand = jnp.full((16,), _BIG, jnp.int32)
                for c, iv in zip(chunks, idxvs):
                    cand = jnp.minimum(cand, jnp.where(c == s, iv, _BIG))
                p = _all_lanes_reduce(cand, jnp.minimum)
                chosen = jnp.where(iota == t, p, chosen)
                chunks = [jnp.where(iv == p, neg, c) for c, iv in zip(chunks, idxvs)]
            idx_v[j, :] = jnp.where(iota < N_SEL, chosen + 1, 0)
        pltpu.sync_copy(idx_v, out_hbm.at[pl.ds(s0, SPW)])

    return k(att_pad)


def _tc_dense(idx, emb_lin, lw, lb, w0, b0, g0, bb0, w1, b1):
    nsl = SLAB * N_SEL

    def issue_wave(idx_ref, emb_hbm, buf, sem, g, slot):
        for s in range(SLAB):
            for n in range(N_SEL):
                row = idx_ref[g * SLAB + s, n] * B + g * SLAB + s
                pltpu.make_async_copy(
                    emb_hbm.at[pl.ds(row, 1)],
                    buf.at[slot, pl.ds(s * N_SEL + n, 1)],
                    sem.at[slot],
                ).start()

    def drain_wave(emb_hbm, buf, sem, slot):
        pltpu.make_async_copy(
            emb_hbm.at[pl.ds(0, nsl)], buf.at[slot], sem.at[slot]).wait()

    def body(idx_ref, emb_hbm, lw_ref, lb_ref, w0_ref, b0_ref, g_ref,
             bb_ref, w1_ref, b1_ref, out_ref, buf, h_acc, cap_acc, sem):
        i = pl.program_id(0)
        slot = lax.rem(i, 2)

        @pl.when(i == 0)
        def _():
            issue_wave(idx_ref, emb_hbm, buf, sem, 0, 0)
            if NSTEP > 1:
                issue_wave(idx_ref, emb_hbm, buf, sem, 1, 1)

        @pl.when((i >= 1) & (i + 1 < NSTEP))
        def _():
            issue_wave(idx_ref, emb_hbm, buf, sem, i + 1, lax.rem(i + 1, 2))

        drain_wave(emb_hbm, buf, sem, slot)

        f = buf[slot]
        n = jnp.sqrt(jnp.sum(f * f, axis=1, keepdims=True)) + 1e-8
        f = f / n
        cap_acc[pl.ds(i * nsl, nsl), :] = lax.dot_general(
            f, lw_ref[...], (((1,), (1,)), ((), ())),
            preferred_element_type=jnp.float32) + lb_ref[...]
        h_acc[pl.ds(i * nsl, nsl), :] = lax.dot_general(
            f, w0_ref[...], (((1,), (1,)), ((), ())),
            preferred_element_type=jnp.float32) + b0_ref[...]

        @pl.when(i == NSTEP - 1)
        def _():
            h = h_acc[...]
            m = jnp.mean(h, axis=0, keepdims=True)
            v = jnp.mean((h - m) ** 2, axis=0, keepdims=True)
            hn = (h - m) / jnp.sqrt(v + 1e-5) * g_ref[...] + bb_ref[...]
            hn = jnp.maximum(hn, 0.0)
            y = lax.dot_general(hn, w1_ref[...], (((1,), (1,)), ((), ())),
                                preferred_element_type=jnp.float32) + b1_ref[...]
            z = (y + cap_acc[...]).reshape(B, N_SEL, D_EMB)
            out_ref[...] = jnp.max(z, axis=1)

    def full(shape):
        return pl.BlockSpec(shape, lambda i, *_: (0,) * len(shape))

    grid_spec = pltpu.PrefetchScalarGridSpec(
        num_scalar_prefetch=1,
        grid=(NSTEP,),
        in_specs=[
            pl.BlockSpec(memory_space=pl.ANY),
            full((D_EMB, D_IN)), full((1, D_EMB)),
            full((HID, D_IN)), full((1, HID)),
            full((1, HID)), full((1, HID)),
            full((D_EMB, HID)), full((1, D_EMB)),
        ],
        out_specs=full((B, D_EMB)),
        scratch_shapes=[
            pltpu.VMEM((2, nsl, D_IN), jnp.float32),
            pltpu.VMEM((B * N_SEL, HID), jnp.float32),
            pltpu.VMEM((B * N_SEL, D_EMB), jnp.float32),
            pltpu.SemaphoreType.DMA((2,)),
        ],
    )
    return pl.pallas_call(
        body,
        grid_spec=grid_spec,
        out_shape=jax.ShapeDtypeStruct((B, D_EMB), jnp.float32),
        compiler_params=pltpu.CompilerParams(
            dimension_semantics=("arbitrary",)),
    )(idx, emb_lin, lw, lb.reshape(1, -1), w0, b0.reshape(1, -1),
      g0.reshape(1, -1), bb0.reshape(1, -1), w1, b1.reshape(1, -1))


def kernel(all_word_embeddings, caption_ids, attention_map, linear_w,
           linear_b, mlp_w0, mlp_b0, bn0_g, bn0_b, mlp_w1, mlp_b1):
    del caption_ids
    att = attention_map[:, L - 1, 1 : L - 1]
    att_pad = jnp.pad(att, ((0, 0), (0, PAD_ATT - N_ATT)),
                      constant_values=_NEG)
    idx = _sc_topk(att_pad)
    emb_lin = jnp.transpose(all_word_embeddings, (1, 0, 2)).reshape(
        L * B, D_IN)
    return _tc_dense(idx, emb_lin, linear_w, linear_b, mlp_w0, mlp_b0,
                     bn0_g, bn0_b, mlp_w1, mlp_b1)


# speedup vs baseline: 1.0137x; 1.0137x over previous
import functools

import jax
import jax.numpy as jnp
from jax import lax
from jax.experimental import pallas as pl
from jax.experimental.pallas import tpu as pltpu
from jax.experimental.pallas import tpu_sc as plsc

B = 128
L = 77
D_IN = 512
D_EMB = 1024
HID = 512
N_SEL = 10
N_ATT = 75
PAD_ATT = 80
NC, NS = 2, 16
NW = NC * NS
SPW = B // NW
SLAB = 64
NSTEP = B // SLAB

_NEG = float("-inf")
_BIG = 2**30


def _lane_rot(x, r):
    perm = (lax.iota(jnp.int32, 16) + r) & 15
    dn = lax.GatherDimensionNumbers(
        offset_dims=(), collapsed_slice_dims=(0,), start_index_map=(0,))
    return lax.gather(x, perm[:, None], dn, slice_sizes=(1,),
                      mode=lax.GatherScatterMode.PROMISE_IN_BOUNDS)


def _all_lanes_reduce(x, op):
    for r in (8, 4, 2, 1):
        x = op(x, _lane_rot(x, r))
    return x


def _sc_topk(att_pad):
    mesh = plsc.VectorSubcoreMesh(core_axis_name="c", subcore_axis_name="s")

    @functools.partial(
        pl.kernel,
        mesh=mesh,
        out_type=jax.ShapeDtypeStruct((B, 16), jnp.int32),
        scratch_types=[
            pltpu.VMEM((SPW, PAD_ATT), jnp.float32),
            pltpu.VMEM((SPW, 16), jnp.int32),
        ],
    )
    def k(att_hbm, out_hbm, att_v, idx_v):
        wid = lax.axis_index("s") * NC + lax.axis_index("c")
        s0 = wid * SPW
        pltpu.sync_copy(att_hbm.at[pl.ds(s0, SPW)], att_v)
        iota = lax.iota(jnp.int32, 16)
        neg = jnp.full((16,), _NEG, jnp.float32)
        for j in range(SPW):
            chunks = [att_v[j, pl.ds(ci * 16, 16)] for ci in range(PAD_ATT // 16)]
            idxvs = [iota + ci * 16 for ci in range(PAD_ATT // 16)]
            chosen = jnp.zeros((16,), jnp.int32)
            for t in range(N_SEL):
                m = chunks[0]
                for c in chunks[1:]:
                    m = jnp.maximum(m, c)
                s = _all_lanes_reduce(m, jnp.maximum)
                cand = jnp.full((16,), _BIG, jnp.int32)
                for c, iv in zip(chunks, idxvs):
                    cand = jnp.minimum(cand, jnp.where(c == s, iv, _BIG))
                p = _all_lanes_reduce(cand, jnp.minimum)
                chosen = jnp.where(iota == t, p, chosen)
                chunks = [jnp.where(iv == p, neg, c) for c, iv in zip(chunks, idxvs)]
            idx_v[j, :] = jnp.where(iota < N_SEL, chosen + 1, 0)
        pltpu.sync_copy(idx_v, out_hbm.at[pl.ds(s0, SPW)])

    return k(att_pad)


def _tc_dense(idx, emb_lin, lw, lb, w0, b0, g0, bb0, w1, b1):
    nsl = SLAB * N_SEL

    def issue_wave(idx_ref, emb_hbm, buf, sem, g, slot):
        for s in range(SLAB):
            for n in range(N_SEL):
                row = idx_ref[g * SLAB + s, n] * B + g * SLAB + s
                pltpu.make_async_copy(
                    emb_hbm.at[pl.ds(row, 1)],
                    buf.at[slot, pl.ds(s * N_SEL + n, 1)],
                    sem.at[slot],
                ).start()

    def drain_wave(emb_hbm, buf, sem, slot):
        pltpu.make_async_copy(
            emb_hbm.at[pl.ds(0, nsl)], buf.at[slot], sem.at[slot]).wait()

    def body(idx_ref, emb_hbm, lw_ref, lb_ref, w0_ref, b0_ref, g_ref,
             bb_ref, w1_ref, b1_ref, out_ref, buf, h_acc, cap_acc, sem):
        i = pl.program_id(0)
        slot = lax.rem(i, 2)

        @pl.when(i == 0)
        def _():
            issue_wave(idx_ref, emb_hbm, buf, sem, 0, 0)
            if NSTEP > 1:
                issue_wave(idx_ref, emb_hbm, buf, sem, 1, 1)

        @pl.when((i >= 1) & (i + 1 < NSTEP))
        def _():
            issue_wave(idx_ref, emb_hbm, buf, sem, i + 1, lax.rem(i + 1, 2))

        drain_wave(emb_hbm, buf, sem, slot)

        f = buf[slot]
        n = jnp.sqrt(jnp.sum(f * f, axis=1, keepdims=True)) + 1e-8
        f = f / n
        cap_acc[pl.ds(i * nsl, nsl), :] = lax.dot_general(
            f, lw_ref[...], (((1,), (1,)), ((), ())),
            preferred_element_type=jnp.float32) + lb_ref[...]
        h_acc[pl.ds(i * nsl, nsl), :] = lax.dot_general(
            f, w0_ref[...], (((1,), (1,)), ((), ())),
            preferred_element_type=jnp.float32) + b0_ref[...]

        @pl.when(i == NSTEP - 1)
        def _():
            h = h_acc[...]
            m = jnp.mean(h, axis=0, keepdims=True)
            v = jnp.mean((h - m) ** 2, axis=0, keepdims=True)
            hn = (h - m) / jnp.sqrt(v + 1e-5) * g_ref[...] + bb_ref[...]
            hn = jnp.maximum(hn, 0.0)
            y = lax.dot_general(hn, w1_ref[...], (((1,), (1,)), ((), ())),
                                preferred_element_type=jnp.float32) + b1_ref[...]
            z = (y + cap_acc[...]).reshape(B, N_SEL, D_EMB)
            out_ref[...] = jnp.max(z, axis=1)

    def full(shape):
        return pl.BlockSpec(shape, lambda i, *_: (0,) * len(shape))

    grid_spec = pltpu.PrefetchScalarGridSpec(
        num_scalar_prefetch=1,
        grid=(NSTEP,),
        in_specs=[
            pl.BlockSpec(memory_space=pl.ANY),
            full((D_EMB, D_IN)), full((1, D_EMB)),
            full((HID, D_IN)), full((1, HID)),
            full((1, HID)), full((1, HID)),
            full((D_EMB, HID)), full((1, D_EMB)),
        ],
        out_specs=full((B, D_EMB)),
        scratch_shapes=[
            pltpu.VMEM((2, nsl, D_IN), jnp.float32),
            pltpu.VMEM((B * N_SEL, HID), jnp.float32),
            pltpu.VMEM((B * N_SEL, D_EMB), jnp.float32),
            pltpu.SemaphoreType.DMA((2,)),
        ],
    )
    return pl.pallas_call(
        body,
        grid_spec=grid_spec,
        out_shape=jax.ShapeDtypeStruct((B, D_EMB), jnp.float32),
        compiler_params=pltpu.CompilerParams(
            dimension_semantics=("arbitrary",)),
    )(idx, emb_lin, lw, lb.reshape(1, -1), w0, b0.reshape(1, -1),
      g0.reshape(1, -1), bb0.reshape(1, -1), w1, b1.reshape(1, -1))


def kernel(all_word_embeddings, caption_ids, attention_map, linear_w,
           linear_b, mlp_w0, mlp_b0, bn0_g, bn0_b, mlp_w1, mlp_b1):
    del caption_ids
    att = attention_map[:, L - 1, 1 : L - 1]
    att_pad = jnp.pad(att, ((0, 0), (0, PAD_ATT - N_ATT)),
                      constant_values=_NEG)
    idx = _sc_topk(att_pad)
    emb_lin = jnp.transpose(all_word_embeddings, (1, 0, 2)).reshape(
        L * B, D_IN)
    return _tc_dense(idx, emb_lin, linear_w, linear_b, mlp_w0, mlp_b0,
                     bn0_g, bn0_b, mlp_w1, mlp_b1)
